# baseline (speedup 1.0000x reference)
from math import factorial, sqrt

import jax
import jax.numpy as jnp
import numpy as np
from jax.experimental import pallas as pl
from jax.experimental.pallas import tpu as pltpu

_L_MAX = 3
_LAMBDA_MAX = 3
_N_ENV = 128
_N_FEAT = 16


def _cg_scalar_np(j1, m1, j2, m2, J, M):
    if m1 + m2 != M:
        return 0.0
    if J < abs(j1 - j2) or J > j1 + j2:
        return 0.0
    pref = sqrt((2 * J + 1) * factorial(J + j1 - j2) * factorial(J - j1 + j2)
                * factorial(j1 + j2 - J) / factorial(j1 + j2 + J + 1))
    pref *= sqrt(factorial(J + M) * factorial(J - M) * factorial(j1 - m1)
                 * factorial(j1 + m1) * factorial(j2 - m2) * factorial(j2 + m2))
    s = 0.0
    for k in range(0, j1 + j2 - J + 1):
        denoms = [k, j1 + j2 - J - k, j1 - m1 - k, j2 + m2 - k,
                  J - j2 + m1 + k, J - j1 - m2 + k]
        if any(d < 0 for d in denoms):
            continue
        term = float((-1) ** k)
        for d in denoms:
            term /= factorial(d)
        s += term
    return pref * s


def _cg_matrix_np(l1, l2, lam):
    mat = np.zeros((2 * l1 + 1, 2 * l2 + 1, 2 * lam + 1), dtype=np.float64)
    for m1 in range(-l1, l1 + 1):
        for m2 in range(-l2, l2 + 1):
            M = m1 + m2
            if -lam <= M <= lam:
                mat[m1 + l1, m2 + l2, M + lam] = _cg_scalar_np(l1, m1, l2, m2, lam, M)
    return np.asarray(mat, dtype=np.float32)


_CGNP = {}
for _l1 in range(_L_MAX + 1):
    for _l2 in range(_L_MAX + 1):
        for _lam in range(abs(_l1 - _l2), min(_l1 + _l2, _LAMBDA_MAX) + 1):
            _CGNP[(_l1, _l2, _lam)] = _cg_matrix_np(_l1, _l2, _lam)

_BLOCKS = {lam: [(l1, l2) for l1 in range(_L_MAX + 1) for l2 in range(_L_MAX + 1)
                 if abs(l1 - l2) <= lam <= l1 + l2]
           for lam in range(_LAMBDA_MAX + 1)}
_NPAIR = {lam: len(_BLOCKS[lam]) for lam in range(_LAMBDA_MAX + 1)}
_ALL_BLOCKS = [(l1, l2, lam) for lam in range(_LAMBDA_MAX + 1)
               for (l1, l2) in _BLOCKS[lam]]

_B_ENV = 8


def _build_w(s, l1, l2, lam):
    c = _CGNP[(l1, l2, lam)]
    O = 2 * lam + 1
    rows = []
    for m in range(2 * l1 + 1):
        cols = []
        for o in range(O):
            n = o - lam - m + l1 + l2
            if 0 <= n < 2 * l2 + 1 and c[m, n, o] != 0.0:
                cols.append(float(c[m, n, o]) * s[:, :, n])
            else:
                cols.append(jnp.zeros((_N_ENV, _N_FEAT), jnp.float32))
        rows.append(jnp.stack(cols, axis=-1).reshape(_N_ENV, 1, _N_FEAT * O))
    return jnp.concatenate(rows, axis=1)


def _tc_body(*refs):
    f_refs = refs[:8]
    w_refs = refs[8:8 + 2 * len(_ALL_BLOCKS)]
    outs = refs[8 + 2 * len(_ALL_BLOCKS):]
    w_by = {}
    for si in range(2):
        for bi, blk in enumerate(_ALL_BLOCKS):
            w_by[(si,) + blk] = w_refs[si * len(_ALL_BLOCKS) + bi]
    for parity in range(2):
        halves = ((0, 0), (4, 1)) if parity == 0 else ((0, 1), (4, 0))
        for lam in range(_LAMBDA_MAX + 1):
            O = 2 * lam + 1
            out_ref = outs[parity * 4 + lam]
            gbase = 0
            for (foff, si) in halves:
                for (l1, l2) in _BLOCKS[lam]:
                    f_ref = f_refs[foff + l1]
                    w_ref = w_by[(si, l1, l2, lam)]
                    g0 = gbase

                    def body(i, _):
                        chunk = None
                        for m in range(2 * l1 + 1):
                            term = f_ref[:, i, m][:, None] * w_ref[:, m, :]
                            chunk = term if chunk is None else chunk + term
                        out_ref[:, g0 + i, :] = chunk
                        return 0

                    jax.lax.fori_loop(0, _N_FEAT, body, 0)
                    gbase += _N_FEAT


def kernel(first_even_0, first_even_1, first_even_2, first_even_3,
           first_odd_0, first_odd_1, first_odd_2, first_odd_3,
           second_even_0, second_even_1, second_even_2, second_even_3,
           second_odd_0, second_odd_1, second_odd_2, second_odd_3):
    fs = (first_even_0, first_even_1, first_even_2, first_even_3,
          first_odd_0, first_odd_1, first_odd_2, first_odd_3)
    se = (second_even_0, second_even_1, second_even_2, second_even_3)
    so = (second_odd_0, second_odd_1, second_odd_2, second_odd_3)
    ws = []
    for second in (se, so):
        for (l1, l2, lam) in _ALL_BLOCKS:
            ws.append(_build_w(second[l2], l1, l2, lam))
    grid = _N_ENV // _B_ENV

    in_specs = [pl.BlockSpec((_B_ENV, _N_FEAT, 2 * (i % 4) + 1),
                             lambda g: (g, 0, 0)) for i in range(8)]
    for _ in range(2):
        for (l1, l2, lam) in _ALL_BLOCKS:
            in_specs.append(pl.BlockSpec(
                (_B_ENV, 2 * l1 + 1, _N_FEAT * (2 * lam + 1)),
                lambda g: (g, 0, 0)))

    out_shapes = []
    out_specs = []
    for parity in range(2):
        for lam in range(_LAMBDA_MAX + 1):
            F = 2 * _N_FEAT * _N_FEAT * _NPAIR[lam]
            G = F // _N_FEAT
            out_shapes.append(jax.ShapeDtypeStruct(
                (_N_ENV, G, _N_FEAT * (2 * lam + 1)), jnp.float32))
            out_specs.append(pl.BlockSpec(
                (_B_ENV, G, _N_FEAT * (2 * lam + 1)), lambda g: (g, 0, 0)))

    outs = pl.pallas_call(
        _tc_body,
        grid=(grid,),
        in_specs=in_specs,
        out_specs=out_specs,
        out_shape=out_shapes,
    )(*fs, *ws)

    res = []
    for parity in range(2):
        for lam in range(_LAMBDA_MAX + 1):
            F = 2 * _N_FEAT * _N_FEAT * _NPAIR[lam]
            res.append(outs[parity * 4 + lam].reshape(_N_ENV, F, 2 * lam + 1))
    return tuple(res)
